# baseline (speedup 1.0000x reference)
import jax
import jax.numpy as jnp
from jax import lax
from jax.experimental import pallas as pl
from jax.experimental.pallas import tpu as pltpu
from jax.experimental.pallas import tpu_sc as plsc

N = 10000
E = 320000
DN = 128
DE = 16
H = 64
L = 4
B = 16

NC = 2
NS = 16
NW = NC * NS
EPAD = 327680
EPW = EPAD // NW
CH = 1024
NCH = EPW // CH
IPC = CH // 128
NA = N + 16

RPT = 640
RLAST = N - RPT * (NS - 1)
ALAST = NA - RPT * (NS - 1)

f32 = jnp.float32

_SC_MESH = plsc.VectorSubcoreMesh(
    core_axis_name="c", subcore_axis_name="s", num_cores=NC, num_subcores=NS)


def _embed_nodes_body(rn_ref, inode_ref, wr_ref, wi_ref, wm0_ref, h0_ref, g0_ref):
    h0 = jnp.dot(rn_ref[...], wr_ref[...], preferred_element_type=f32)
    h0 = h0 + inode_ref[...] * wi_ref[...]
    h0_ref[...] = h0
    g0_ref[...] = jnp.dot(h0, wm0_ref[...], preferred_element_type=f32)


def _embed_edges_body(r0_, r1_, r2_, r3_, r4_, r5_, r6_, r7_,
                      wre_ref, wmsg_ref, o0, o1, o2, o3):
    subs = (r0_, r1_, r2_, r3_, r4_, r5_, r6_, r7_)
    for l, o in enumerate((o0, o1, o2, o3)):
        wc = jnp.dot(wre_ref[...], wmsg_ref[l], preferred_element_type=f32)
        o[...] = jnp.concatenate(
            [jnp.dot(sub[...], wc, preferred_element_type=f32)
             for sub in subs], axis=1)


def _layer_update_body(h_ref, agg_ref, ws_ref, wm_ref, hn_ref, gn_ref):
    a = agg_ref[0] + agg_ref[1]
    t = jnp.dot(a, ws_ref[...], preferred_element_type=f32)
    t = jnp.maximum(t, 0.01 * t)
    hn = h_ref[...] + 0.1 * t
    hn_ref[...] = hn
    gn_ref[...] = jnp.dot(hn, wm_ref[...], preferred_element_type=f32)


def _layer_final_body(h_ref, agg_ref, ws_ref, de_ref, wo_ref, p_ref):
    a = agg_ref[0] + agg_ref[1]
    t = jnp.dot(a, ws_ref[...], preferred_element_type=f32)
    t = jnp.maximum(t, 0.01 * t)
    hn = h_ref[...] + 0.1 * t
    p_ref[...] = jnp.dot(hn * de_ref[...], wo_ref[...], preferred_element_type=f32)


def _tail_body(part_ref, g_ref, b_ref, lw_ref, lb_ref, out_ref):
    pooled = jnp.sum(part_ref[...], axis=(0, 2)).reshape(B, 1)
    mu = jnp.mean(pooled, axis=-1, keepdims=True)
    var = jnp.var(pooled, axis=-1, keepdims=True)
    normed = (pooled - mu) / jnp.sqrt(var + 1e-5) * g_ref[...] + b_ref[...]
    out_ref[...] = jnp.tanh(normed) * lw_ref[...] + lb_ref[...]


def _edge_pass_body(g_hbm, em_hbm, srcm_hbm, dstm_hbm, agg_out,
                    g_sh, acc_sh, em0, em1, gat0, gat1, gat2, src_v,
                    dst_v, se0, se1, sg0, sg1, sg2):
    ems = (em0, em1)
    gats = (gat0, gat1, gat2)
    sems_e = (se0, se1)
    sems_g = (sg0, sg1, sg2)
    c = lax.axis_index("c")
    s = lax.axis_index("s")
    w = s * NC + c

    def zbody(r, _):
        for q in range(4):
            gat0[r, pl.ds(q * 16, 16)] = jnp.zeros((16,), f32)
        return 0
    lax.fori_loop(0, 128, zbody, 0)

    r0 = s * RPT

    @pl.when(s < NS - 1)
    def _():
        for t in range(5):
            pltpu.sync_copy(gat0, acc_sh.at[pl.ds(r0 + t * 128, 128)])
        for t in range(5):
            pltpu.sync_copy(g_hbm.at[pl.ds(r0 + t * 128, 128)], gat1)
            pltpu.sync_copy(gat1, g_sh.at[pl.ds(r0 + t * 128, 128)])

    @pl.when(s == NS - 1)
    def _():
        for t in (0, 128, 256, 288):
            pltpu.sync_copy(gat0, acc_sh.at[pl.ds(r0 + t, 128)])
        for t in (0, 128, 256, 272):
            pltpu.sync_copy(g_hbm.at[pl.ds(r0 + t, 128)], gat1)
            pltpu.sync_copy(gat1, g_sh.at[pl.ds(r0 + t, 128)])

    plsc.subcore_barrier()

    def compute(gat, em_ref):
        def cbody(r, _):
            for q in range(4):
                sl = pl.ds(q * 16, 16)
                x = gat[r, sl] + em_ref[r, sl]
                gat[r, sl] = jnp.maximum(x, 0.01 * x)
            return 0
        lax.fori_loop(0, 128, cbody, 0)

    def chunk(k, _):
        irow = w * (EPW // 128) + k * IPC
        pltpu.sync_copy(srcm_hbm.at[pl.ds(irow, IPC)], src_v)
        pltpu.sync_copy(dstm_hbm.at[pl.ds(irow, IPC)], dst_v)
        nbase = w * EPW + k * CH

        def em_copy(j, buf, sem):
            n0 = nbase + j * 128
            row = (n0 // 4096) * 512 + n0 % 512
            lane = ((n0 % 4096) // 512) * H
            return pltpu.async_copy(
                em_hbm.at[pl.ds(row, 128), pl.ds(lane, H)], buf, sem)

        de = {}
        for j in range(2):
            de[j] = em_copy(j, ems[j % 2], sems_e[j % 2])
        d = {}
        for j in range(3):
            d[j] = pltpu.async_copy(
                g_sh.at[src_v.at[j]], gats[j % 3], sems_g[j % 3])
        sd = {}
        for j in range(8):
            gbuf = gats[j % 3]
            de[j].wait()
            d[j].wait()
            compute(gbuf, ems[j % 2])
            sd[j] = pltpu.async_copy(gbuf, acc_sh.at[dst_v.at[j]],
                                     sems_g[j % 3], add=True)
            if j + 2 <= 7:
                de[j + 2] = em_copy(j + 2, ems[(j + 2) % 2],
                                    sems_e[(j + 2) % 2])
            if 1 <= j <= 5:
                sd[j - 1].wait()
                d[j + 2] = pltpu.async_copy(
                    g_sh.at[src_v.at[j + 2]], gats[(j + 2) % 3],
                    sems_g[(j + 2) % 3])
        sd[5].wait()
        sd[6].wait()
        sd[7].wait()
        return 0
    lax.fori_loop(0, NCH, chunk, 0)

    plsc.subcore_barrier()

    @pl.when(s < NS - 1)
    def _():
        for t in range(5):
            pltpu.sync_copy(acc_sh.at[pl.ds(r0 + t * 128, 128)], gat0)
            pltpu.sync_copy(gat0, agg_out.at[c, pl.ds(r0 + t * 128, 128)])

    @pl.when(s == NS - 1)
    def _():
        for t in (0, 128, 256, 272):
            pltpu.sync_copy(acc_sh.at[pl.ds(r0 + t, 128)], gat0)
            pltpu.sync_copy(gat0, agg_out.at[c, pl.ds(r0 + t, 128)])


_edge_pass = pl.kernel(
    _edge_pass_body,
    out_type=jax.ShapeDtypeStruct((NC, N, H), f32),
    mesh=_SC_MESH,
    compiler_params=pltpu.CompilerParams(
        use_tc_tiling_on_sc=False, needs_layout_passes=False),
    scratch_types=[
        pltpu.VMEM_SHARED((N, H), f32),
        pltpu.VMEM_SHARED((NA, H), f32),
        pltpu.VMEM((128, H), f32),
        pltpu.VMEM((128, H), f32),
        pltpu.VMEM((128, H), f32),
        pltpu.VMEM((128, H), f32),
        pltpu.VMEM((128, H), f32),
        pltpu.VMEM((IPC, 128), jnp.int32),
        pltpu.VMEM((IPC, 128), jnp.int32),
        pltpu.SemaphoreType.DMA,
        pltpu.SemaphoreType.DMA,
        pltpu.SemaphoreType.DMA,
        pltpu.SemaphoreType.DMA,
        pltpu.SemaphoreType.DMA,
    ],
)


_PPW = 320
_PLAST = N - _PPW * (NW - 1)


def _bin_pool_body(p_hbm, r2d_hbm, dseg_hbm, part_out, p_v, r_v, dseg_v, acc_v):
    c = lax.axis_index("c")
    s = lax.axis_index("s")
    w = s * NC + c
    pltpu.sync_copy(dseg_hbm, dseg_v)
    for b in range(B):
        acc_v[b, pl.ds(0, 16)] = jnp.zeros((16,), f32)

    base = w * _PPW
    lane = lax.iota(jnp.int32, 16)

    def run(cnt):
        pltpu.sync_copy(p_hbm.at[pl.ds(base, cnt)], p_v.at[pl.ds(0, cnt)])
        pltpu.sync_copy(r2d_hbm.at[pl.ds(base, cnt)], r_v.at[pl.ds(0, cnt)])

        def gbody(j, _):
            idx = r_v[pl.ds(j * 16, 16)]
            seg = plsc.load_gather(dseg_v, [idx])
            pv = p_v[pl.ds(j * 16, 16)]
            plsc.addupdate_scatter(acc_v, [seg, lane], pv)
            return 0
        lax.fori_loop(0, cnt // 16, gbody, 0)

    @pl.when(w < NW - 1)
    def _():
        run(_PPW)

    @pl.when(w == NW - 1)
    def _():
        run(_PLAST)

    pltpu.sync_copy(acc_v, part_out.at[w])


_bin_pool = pl.kernel(
    _bin_pool_body,
    out_type=jax.ShapeDtypeStruct((NW, B, 16), f32),
    mesh=_SC_MESH,
    compiler_params=pltpu.CompilerParams(
        use_tc_tiling_on_sc=False, needs_layout_passes=False),
    scratch_types=[
        pltpu.VMEM((_PPW,), f32),
        pltpu.VMEM((_PPW,), jnp.int32),
        pltpu.VMEM((N,), jnp.int32),
        pltpu.VMEM((B, 16), f32),
    ],
)


_EBLK = 8192


def kernel(r_node, i_node, r_edge, d_edge, edge_index, r2d_dst, d_segment_ids,
           W_rnode, W_inode, W_redge, W_msg, W_self, W_out,
           ln_gamma, ln_beta, lin_w, lin_b):
    sds = jax.ShapeDtypeStruct
    npad = EPAD - E


    pad_src = jnp.zeros((npad,), jnp.int32)
    pad_dst = N + (jnp.arange(npad, dtype=jnp.int32) % 16)
    srcm = jnp.concatenate([edge_index[0], pad_src]).reshape(EPAD // 128, 128)
    dstm = jnp.concatenate([edge_index[1], pad_dst]).reshape(EPAD // 128, 128)

    h, g = pl.pallas_call(
        _embed_nodes_body,
        out_shape=[sds((N, H), f32), sds((N, H), f32)],
    )(r_node, i_node, W_rnode, W_inode, W_msg[0])

    def _sub_spec(e):
        return pl.BlockSpec(
            (512, DE), lambda i, e=e: (jnp.minimum(8 * i + e, E // 512 - 1), 0))

    em = pl.pallas_call(
        _embed_edges_body,
        grid=(EPAD // 4096,),
        in_specs=[_sub_spec(e) for e in range(8)] + [
            pl.BlockSpec((DE, H), lambda i: (0, 0)),
            pl.BlockSpec((L, H, H), lambda i: (0, 0, 0)),
        ],
        out_specs=[pl.BlockSpec((512, 8 * H), lambda i: (i, 0))] * L,
        out_shape=[sds((EPAD // 8, 8 * H), f32)] * L,
    )(*([r_edge] * 8), W_redge, W_msg)

    for l in range(L):
        agg = _edge_pass(g, em[l], srcm, dstm)
        if l < L - 1:
            h, g = pl.pallas_call(
                _layer_update_body,
                out_shape=[sds((N, H), f32), sds((N, H), f32)],
            )(h, agg, W_self[l], W_msg[l + 1])
        else:
            p = pl.pallas_call(
                _layer_final_body,
                out_shape=sds((N, 1), f32),
            )(h, agg, W_self[l], d_edge, W_out)

    part = _bin_pool(p.reshape(N), r2d_dst, d_segment_ids)

    out = pl.pallas_call(
        _tail_body,
        out_shape=sds((B, 1), f32),
    )(part, ln_gamma.reshape(1, 1), ln_beta.reshape(1, 1), lin_w,
      lin_b.reshape(1, 1))
    return out


# speedup vs baseline: 1.0210x; 1.0210x over previous
import jax
import jax.numpy as jnp
from jax import lax
from jax.experimental import pallas as pl
from jax.experimental.pallas import tpu as pltpu
from jax.experimental.pallas import tpu_sc as plsc

N = 10000
E = 320000
DN = 128
DE = 16
H = 64
L = 4
B = 16

NC = 2
NS = 16
NW = NC * NS
EPAD = 327680
EPW = EPAD // NW
CH = 1024
NCH = EPW // CH
IPC = CH // 128
NA = N + 16

RPT = 640
RLAST = N - RPT * (NS - 1)
ALAST = NA - RPT * (NS - 1)

f32 = jnp.float32

_SC_MESH = plsc.VectorSubcoreMesh(
    core_axis_name="c", subcore_axis_name="s", num_cores=NC, num_subcores=NS)


def _embed_nodes_body(rn_ref, inode_ref, wr_ref, wi_ref, wm0_ref, h0_ref, g0_ref):
    h0 = jnp.dot(rn_ref[...], wr_ref[...], preferred_element_type=f32)
    h0 = h0 + inode_ref[...] * wi_ref[...]
    h0_ref[...] = h0
    g0_ref[...] = jnp.dot(h0, wm0_ref[...], preferred_element_type=f32)


def _embed_edges_body(re_ref, wre_ref, wmsg_ref, o0, o1, o2, o3):
    blk = re_ref[...]
    z = jnp.zeros((DE, H), f32)
    for l, o in enumerate((o0, o1, o2, o3)):
        wc = jnp.dot(wre_ref[...], wmsg_ref[l], preferred_element_type=f32)
        rows = [jnp.concatenate([z] * e + [wc] + [z] * (7 - e), axis=1)
                for e in range(8)]
        bd = jnp.concatenate(rows, axis=0)
        o[...] = jnp.dot(blk, bd, preferred_element_type=f32)


def _layer_update_body(h_ref, agg_ref, ws_ref, wm_ref, hn_ref, gn_ref):
    a = agg_ref[0] + agg_ref[1]
    t = jnp.dot(a, ws_ref[...], preferred_element_type=f32)
    t = jnp.maximum(t, 0.01 * t)
    hn = h_ref[...] + 0.1 * t
    hn_ref[...] = hn
    gn_ref[...] = jnp.dot(hn, wm_ref[...], preferred_element_type=f32)


def _layer_final_body(h_ref, agg_ref, ws_ref, de_ref, wo_ref, p_ref):
    a = agg_ref[0] + agg_ref[1]
    t = jnp.dot(a, ws_ref[...], preferred_element_type=f32)
    t = jnp.maximum(t, 0.01 * t)
    hn = h_ref[...] + 0.1 * t
    p_ref[...] = jnp.dot(hn * de_ref[...], wo_ref[...], preferred_element_type=f32)


def _tail_body(part_ref, g_ref, b_ref, lw_ref, lb_ref, out_ref):
    pooled = jnp.sum(part_ref[...], axis=(0, 2)).reshape(B, 1)
    mu = jnp.mean(pooled, axis=-1, keepdims=True)
    var = jnp.var(pooled, axis=-1, keepdims=True)
    normed = (pooled - mu) / jnp.sqrt(var + 1e-5) * g_ref[...] + b_ref[...]
    out_ref[...] = jnp.tanh(normed) * lw_ref[...] + lb_ref[...]


def _edge_pass_body(g_hbm, em_hbm, srcm_hbm, dstm_hbm, agg_out,
                    g_sh, acc_sh, em0, em1, gat0, gat1, gat2, src_v,
                    dst_v, se0, se1, sg0, sg1, sg2):
    ems = (em0, em1)
    gats = (gat0, gat1, gat2)
    sems_e = (se0, se1)
    sems_g = (sg0, sg1, sg2)
    c = lax.axis_index("c")
    s = lax.axis_index("s")
    w = s * NC + c

    def zbody(r, _):
        for q in range(4):
            gat0[r, pl.ds(q * 16, 16)] = jnp.zeros((16,), f32)
        return 0
    lax.fori_loop(0, 128, zbody, 0)

    r0 = s * RPT

    @pl.when(s < NS - 1)
    def _():
        for t in range(5):
            pltpu.sync_copy(gat0, acc_sh.at[pl.ds(r0 + t * 128, 128)])
        for t in range(5):
            pltpu.sync_copy(g_hbm.at[pl.ds(r0 + t * 128, 128)], gat1)
            pltpu.sync_copy(gat1, g_sh.at[pl.ds(r0 + t * 128, 128)])

    @pl.when(s == NS - 1)
    def _():
        for t in (0, 128, 256, 288):
            pltpu.sync_copy(gat0, acc_sh.at[pl.ds(r0 + t, 128)])
        for t in (0, 128, 256, 272):
            pltpu.sync_copy(g_hbm.at[pl.ds(r0 + t, 128)], gat1)
            pltpu.sync_copy(gat1, g_sh.at[pl.ds(r0 + t, 128)])

    plsc.subcore_barrier()

    def compute(gat, em_ref):
        def cbody(r, _):
            for q in range(4):
                x = (gat[r, pl.ds(q * 16, 16)]
                     + em_ref[pl.ds(r * H + q * 16, 16)])
                gat[r, pl.ds(q * 16, 16)] = jnp.maximum(x, 0.01 * x)
            return 0
        lax.fori_loop(0, 128, cbody, 0)

    def chunk(k, _):
        irow = w * (EPW // 128) + k * IPC
        pltpu.sync_copy(srcm_hbm.at[pl.ds(irow, IPC)], src_v)
        pltpu.sync_copy(dstm_hbm.at[pl.ds(irow, IPC)], dst_v)
        ebase = (w * EPW + k * CH) * H
        de = {}
        for j in range(2):
            de[j] = pltpu.async_copy(
                em_hbm.at[pl.ds(ebase + j * 8192, 8192)], ems[j % 2],
                sems_e[j % 2])
        d = {}
        for j in range(3):
            d[j] = pltpu.async_copy(
                g_sh.at[src_v.at[j]], gats[j % 3], sems_g[j % 3])
        sd = {}
        for j in range(8):
            gbuf = gats[j % 3]
            de[j].wait()
            d[j].wait()
            compute(gbuf, ems[j % 2])
            sd[j] = pltpu.async_copy(gbuf, acc_sh.at[dst_v.at[j]],
                                     sems_g[j % 3], add=True)
            if j + 2 <= 7:
                de[j + 2] = pltpu.async_copy(
                    em_hbm.at[pl.ds(ebase + (j + 2) * 8192, 8192)],
                    ems[(j + 2) % 2], sems_e[(j + 2) % 2])
            if 1 <= j <= 5:
                sd[j - 1].wait()
                d[j + 2] = pltpu.async_copy(
                    g_sh.at[src_v.at[j + 2]], gats[(j + 2) % 3],
                    sems_g[(j + 2) % 3])
        sd[5].wait()
        sd[6].wait()
        sd[7].wait()
        return 0
    lax.fori_loop(0, NCH, chunk, 0)

    plsc.subcore_barrier()

    @pl.when(s < NS - 1)
    def _():
        for t in range(5):
            pltpu.sync_copy(acc_sh.at[pl.ds(r0 + t * 128, 128)], gat0)
            pltpu.sync_copy(gat0, agg_out.at[c, pl.ds(r0 + t * 128, 128)])

    @pl.when(s == NS - 1)
    def _():
        for t in (0, 128, 256, 272):
            pltpu.sync_copy(acc_sh.at[pl.ds(r0 + t, 128)], gat0)
            pltpu.sync_copy(gat0, agg_out.at[c, pl.ds(r0 + t, 128)])


_edge_pass = pl.kernel(
    _edge_pass_body,
    out_type=jax.ShapeDtypeStruct((NC, N, H), f32),
    mesh=_SC_MESH,
    compiler_params=pltpu.CompilerParams(
        use_tc_tiling_on_sc=False, needs_layout_passes=False),
    scratch_types=[
        pltpu.VMEM_SHARED((N, H), f32),
        pltpu.VMEM_SHARED((NA, H), f32),
        pltpu.VMEM((8192,), f32),
        pltpu.VMEM((8192,), f32),
        pltpu.VMEM((128, H), f32),
        pltpu.VMEM((128, H), f32),
        pltpu.VMEM((128, H), f32),
        pltpu.VMEM((IPC, 128), jnp.int32),
        pltpu.VMEM((IPC, 128), jnp.int32),
        pltpu.SemaphoreType.DMA,
        pltpu.SemaphoreType.DMA,
        pltpu.SemaphoreType.DMA,
        pltpu.SemaphoreType.DMA,
        pltpu.SemaphoreType.DMA,
    ],
)


_PPW = 320
_PLAST = N - _PPW * (NW - 1)


def _bin_pool_body(p_hbm, r2d_hbm, dseg_hbm, part_out, p_v, r_v, dseg_v, acc_v):
    c = lax.axis_index("c")
    s = lax.axis_index("s")
    w = s * NC + c
    pltpu.sync_copy(dseg_hbm, dseg_v)
    for b in range(B):
        acc_v[b, pl.ds(0, 16)] = jnp.zeros((16,), f32)

    base = w * _PPW
    lane = lax.iota(jnp.int32, 16)

    def run(cnt):
        pltpu.sync_copy(p_hbm.at[pl.ds(base, cnt)], p_v.at[pl.ds(0, cnt)])
        pltpu.sync_copy(r2d_hbm.at[pl.ds(base, cnt)], r_v.at[pl.ds(0, cnt)])

        def gbody(j, _):
            idx = r_v[pl.ds(j * 16, 16)]
            seg = plsc.load_gather(dseg_v, [idx])
            pv = p_v[pl.ds(j * 16, 16)]
            plsc.addupdate_scatter(acc_v, [seg, lane], pv)
            return 0
        lax.fori_loop(0, cnt // 16, gbody, 0)

    @pl.when(w < NW - 1)
    def _():
        run(_PPW)

    @pl.when(w == NW - 1)
    def _():
        run(_PLAST)

    pltpu.sync_copy(acc_v, part_out.at[w])


_bin_pool = pl.kernel(
    _bin_pool_body,
    out_type=jax.ShapeDtypeStruct((NW, B, 16), f32),
    mesh=_SC_MESH,
    compiler_params=pltpu.CompilerParams(
        use_tc_tiling_on_sc=False, needs_layout_passes=False),
    scratch_types=[
        pltpu.VMEM((_PPW,), f32),
        pltpu.VMEM((_PPW,), jnp.int32),
        pltpu.VMEM((N,), jnp.int32),
        pltpu.VMEM((B, 16), f32),
    ],
)


_EBLK = 8192


def kernel(r_node, i_node, r_edge, d_edge, edge_index, r2d_dst, d_segment_ids,
           W_rnode, W_inode, W_redge, W_msg, W_self, W_out,
           ln_gamma, ln_beta, lin_w, lin_b):
    sds = jax.ShapeDtypeStruct
    npad = EPAD - E


    pad_src = jnp.zeros((npad,), jnp.int32)
    pad_dst = N + (jnp.arange(npad, dtype=jnp.int32) % 16)
    srcm = jnp.concatenate([edge_index[0], pad_src]).reshape(EPAD // 128, 128)
    dstm = jnp.concatenate([edge_index[1], pad_dst]).reshape(EPAD // 128, 128)

    h, g = pl.pallas_call(
        _embed_nodes_body,
        out_shape=[sds((N, H), f32), sds((N, H), f32)],
    )(r_node, i_node, W_rnode, W_inode, W_msg[0])

    re_pad = jnp.concatenate(
        [r_edge.astype(f32).reshape(E // 8, 8 * DE),
         jnp.zeros((npad // 8, 8 * DE), f32)], axis=0)

    em = pl.pallas_call(
        _embed_edges_body,
        grid=(EPAD // _EBLK,),
        in_specs=[
            pl.BlockSpec((_EBLK // 8, 8 * DE), lambda i: (i, 0)),
            pl.BlockSpec((DE, H), lambda i: (0, 0)),
            pl.BlockSpec((L, H, H), lambda i: (0, 0, 0)),
        ],
        out_specs=[pl.BlockSpec((_EBLK // 8, 8 * H), lambda i: (i, 0))] * L,
        out_shape=[sds((EPAD // 8, 8 * H), f32)] * L,
    )(re_pad, W_redge, W_msg)

    for l in range(L):
        agg = _edge_pass(g, em[l].reshape(-1), srcm, dstm)
        if l < L - 1:
            h, g = pl.pallas_call(
                _layer_update_body,
                out_shape=[sds((N, H), f32), sds((N, H), f32)],
            )(h, agg, W_self[l], W_msg[l + 1])
        else:
            p = pl.pallas_call(
                _layer_final_body,
                out_shape=sds((N, 1), f32),
            )(h, agg, W_self[l], d_edge, W_out)

    part = _bin_pool(p.reshape(N), r2d_dst, d_segment_ids)

    out = pl.pallas_call(
        _tail_body,
        out_shape=sds((B, 1), f32),
    )(part, ln_gamma.reshape(1, 1), ln_beta.reshape(1, 1), lin_w,
      lin_b.reshape(1, 1))
    return out
